# Initial kernel scaffold; baseline (speedup 1.0000x reference)
import jax
import jax.numpy as jnp
from jax.experimental import pallas as pl


def kernel(hidden_states, gate_w, expert_bias, w_gate_up, w_down, shared_gate_up, shared_down):
    raise NotImplementedError("write your pallas kernel here")


# speedup vs baseline: 3.0015x; 3.0015x over previous
import jax
import jax.numpy as jnp
from jax.experimental import pallas as pl
from jax.experimental.pallas import tpu as pltpu

_E = 64
_K = 8
_D = 1024
_DFF = 256
_T = 2048
_TB = 256


def _routing_body(logits_ref, bias_ref, w_ref):
    logits = logits_ref[...]
    scores = jax.nn.sigmoid(logits)
    choice = scores + bias_ref[...]
    c1 = choice[:, :, None]
    c2 = choice[:, None, :]
    ii = jax.lax.broadcasted_iota(jnp.int32, (1, _E, _E), 1)
    jj = jax.lax.broadcasted_iota(jnp.int32, (1, _E, _E), 2)
    beats = (c2 > c1) | ((c2 == c1) & (jj < ii))
    rank = jnp.sum(beats.astype(jnp.float32), axis=2)
    sel = (rank < float(_K)).astype(jnp.float32)
    w = scores * sel
    w = w / jnp.sum(w, axis=1, keepdims=True)
    w_ref[...] = w


def _moe_body(x_ref, wgu_ref, wd_ref, w_ref, sgu_ref, sdn_ref, out_ref):
    e = pl.program_id(0)
    x = x_ref[...]
    gu = jax.lax.dot_general(
        x, wgu_ref[0], (((1,), (1,)), ((), ())),
        preferred_element_type=jnp.float32)
    g = gu[:, :_DFF]
    u = gu[:, _DFF:]
    h = (jax.nn.silu(g) * u).astype(jnp.bfloat16)
    ye = jax.lax.dot_general(
        h, wd_ref[0], (((1,), (1,)), ((), ())),
        preferred_element_type=jnp.float32)
    w_all = w_ref[...]
    lane = jax.lax.broadcasted_iota(jnp.int32, (1, _E), 1)
    wcol = jnp.sum(w_all * (lane == e).astype(jnp.float32), axis=1,
                   keepdims=True)
    contrib = ye * wcol

    @pl.when(e == 0)
    def _():
        out_ref[...] = jnp.zeros_like(out_ref)

    out_ref[...] += contrib

    @pl.when(e == _E - 1)
    def _():
        gu2 = jax.lax.dot_general(
            x, sgu_ref[...], (((1,), (1,)), ((), ())),
            preferred_element_type=jnp.float32)
        nsh = sgu_ref.shape[0] // 2
        h2 = (jax.nn.silu(gu2[:, :nsh]) * gu2[:, nsh:]).astype(jnp.bfloat16)
        sh = jax.lax.dot_general(
            h2, sdn_ref[...], (((1,), (1,)), ((), ())),
            preferred_element_type=jnp.float32)
        out_ref[...] += sh


def kernel(hidden_states, gate_w, expert_bias, w_gate_up, w_down,
           shared_gate_up, shared_down):
    logits = hidden_states.astype(jnp.float32) @ gate_w.astype(jnp.float32).T
    w = pl.pallas_call(
        _routing_body,
        grid=(_T // _TB,),
        in_specs=[
            pl.BlockSpec((_TB, _E), lambda t: (t, 0)),
            pl.BlockSpec((1, _E), lambda t: (0, 0)),
        ],
        out_specs=pl.BlockSpec((_TB, _E), lambda t: (t, 0)),
        out_shape=jax.ShapeDtypeStruct((_T, _E), jnp.float32),
    )(logits, expert_bias.reshape(1, _E))

    xb = hidden_states.astype(jnp.bfloat16)
    out = pl.pallas_call(
        _moe_body,
        grid=(_E,),
        in_specs=[
            pl.BlockSpec((_T, _D), lambda e: (0, 0)),
            pl.BlockSpec((1, 2 * _DFF, _D), lambda e: (e, 0, 0)),
            pl.BlockSpec((1, _D, _DFF), lambda e: (e, 0, 0)),
            pl.BlockSpec((_T, _E), lambda e: (0, 0)),
            pl.BlockSpec(shared_gate_up.shape, lambda e: (0, 0)),
            pl.BlockSpec(shared_down.shape, lambda e: (0, 0)),
        ],
        out_specs=pl.BlockSpec((_T, _D), lambda e: (0, 0)),
        out_shape=jax.ShapeDtypeStruct((_T, _D), jnp.float32),
        compiler_params=pltpu.CompilerParams(
            dimension_semantics=("arbitrary",)),
    )(xb, w_gate_up.astype(jnp.bfloat16), w_down.astype(jnp.bfloat16), w,
      shared_gate_up.astype(jnp.bfloat16), shared_down.astype(jnp.bfloat16))
    return out
